# baseline (device time: 118585 ns/iter reference)
import jax
import jax.numpy as jnp
from jax import lax
from jax.experimental import pallas as pl
from jax.experimental.pallas import tpu as pltpu

N_DEV = 8
B, SQ, D_MODEL, HQ, DH = 2, 512, 768, 8, 64
BLK = 64
NEG = -1e9


def kernel(x, Wq, K_ext, V_ext, Wo):
    def body(x_ref, wq_ref, k_ref, v_ref, wo_ref, out_ref,
             ctx_ref, send_sem, recv_sem):
        my = lax.axis_index("i")
        left = (my + N_DEV - 1) % N_DEV
        right = (my + 1) % N_DEV

        barrier_sem = pltpu.get_barrier_semaphore()
        for nbr in (left, right):
            pl.semaphore_signal(
                barrier_sem, inc=1,
                device_id=(nbr,), device_id_type=pl.DeviceIdType.MESH,
            )
        pl.semaphore_wait(barrier_sem, 2)

        @pl.when(my == 0)
        def _compute():
            qb = lax.broadcasted_iota(jnp.int32, (SQ, SQ), 0) // BLK
            kb = lax.broadcasted_iota(jnp.int32, (SQ, SQ), 1) // BLK
            mask = kb <= qb
            for b in range(B):
                q_b = jnp.dot(x_ref[b], wq_ref[...],
                              preferred_element_type=jnp.float32)
                k_b = k_ref[b]
                v_b = v_ref[b]
                for h in range(HQ):
                    q = q_b[:, h * DH:(h + 1) * DH]
                    s = lax.dot_general(
                        q, k_b[:, h, :], (((1,), (1,)), ((), ())),
                        preferred_element_type=jnp.float32) * 0.125
                    s = jnp.where(mask, s, NEG)
                    m = jnp.max(s, axis=-1, keepdims=True)
                    w = jnp.exp(s - m)
                    w = w / jnp.sum(w, axis=-1, keepdims=True)
                    ctx_ref[b, :, h * DH:(h + 1) * DH] = jnp.dot(
                        w, v_b[:, h, :], preferred_element_type=jnp.float32)

        rdma = pltpu.make_async_remote_copy(
            src_ref=ctx_ref,
            dst_ref=ctx_ref,
            send_sem=send_sem,
            recv_sem=recv_sem,
            device_id=(right,),
            device_id_type=pl.DeviceIdType.MESH,
        )

        @pl.when(my == 0)
        def _send0():
            rdma.start()
            rdma.wait_send()

        @pl.when(my > 0)
        def _recv():
            rdma.wait_recv()

        @pl.when(jnp.logical_and(my > 0, my < N_DEV - 1))
        def _fwd():
            rdma.start()
            rdma.wait_send()

        for b in range(B):
            out_ref[b] = jnp.dot(ctx_ref[b], wo_ref[...],
                                 preferred_element_type=jnp.float32)

    return pl.pallas_call(
        body,
        out_shape=jax.ShapeDtypeStruct((B, SQ, D_MODEL), jnp.float32),
        in_specs=[pl.BlockSpec(memory_space=pltpu.VMEM)] * 5,
        out_specs=pl.BlockSpec(memory_space=pltpu.VMEM),
        scratch_shapes=[
            pltpu.VMEM((B, SQ, HQ * DH), jnp.float32),
            pltpu.SemaphoreType.DMA,
            pltpu.SemaphoreType.DMA,
        ],
        compiler_params=pltpu.CompilerParams(collective_id=0),
    )(x, Wq, K_ext, V_ext, Wo)


# device time: 48944 ns/iter; 2.4229x vs baseline; 2.4229x over previous
import jax
import jax.numpy as jnp
from jax import lax
from jax.experimental import pallas as pl
from jax.experimental.pallas import tpu as pltpu

N_DEV = 8
B, SQ, D_MODEL, HQ, DH = 2, 512, 768, 8, 64
BLK = 64
NEG = -1e9

CH = 128
NCH_B = SQ // CH
NCH = B * NCH_B


def kernel(x, Wq, K_ext, V_ext, Wo):
    def body(x_ref, wq_ref, k_ref, v_ref, wo_ref, out_ref,
             ctx_ref, send_sems, recv_sems):
        my = lax.axis_index("i")
        left = (my + N_DEV - 1) % N_DEV
        right = (my + 1) % N_DEV

        barrier_sem = pltpu.get_barrier_semaphore()
        for nbr in (left, right):
            pl.semaphore_signal(
                barrier_sem, inc=1,
                device_id=(nbr,), device_id_type=pl.DeviceIdType.MESH,
            )
        pl.semaphore_wait(barrier_sem, 2)

        rdmas = []
        for ci in range(NCH):
            b, c = divmod(ci, NCH_B)
            chunk = (b, pl.ds(c * CH, CH), slice(None))
            rdmas.append(pltpu.make_async_remote_copy(
                src_ref=ctx_ref.at[chunk],
                dst_ref=ctx_ref.at[chunk],
                send_sem=send_sems.at[ci],
                recv_sem=recv_sems.at[ci],
                device_id=(right,),
                device_id_type=pl.DeviceIdType.MESH,
            ))

        def project(ci):
            b, c = divmod(ci, NCH_B)
            rows = pl.ds(c * CH, CH)
            out_ref[b, rows, :] = jnp.dot(
                ctx_ref[b, rows, :], wo_ref[...],
                preferred_element_type=jnp.float32)

        @pl.when(my == 0)
        def _producer():
            for b in range(B):
                q_b = jnp.dot(x_ref[b], wq_ref[...],
                              preferred_element_type=jnp.float32)
                k_b = k_ref[b]
                v_b = v_ref[b]
                for c in range(NCH_B):
                    ci = b * NCH_B + c
                    kmax = (c + 1) * CH
                    rows = slice(c * CH, (c + 1) * CH)
                    qblk = (lax.broadcasted_iota(jnp.int32, (CH, kmax), 0)
                            + c * CH) // BLK
                    kblk = lax.broadcasted_iota(jnp.int32, (CH, kmax), 1) // BLK
                    mask = kblk <= qblk
                    for h in range(HQ):
                        q = q_b[rows, h * DH:(h + 1) * DH]
                        s = lax.dot_general(
                            q, k_b[:kmax, h, :], (((1,), (1,)), ((), ())),
                            preferred_element_type=jnp.float32) * 0.125
                        s = jnp.where(mask, s, NEG)
                        m = jnp.max(s, axis=-1, keepdims=True)
                        w = jnp.exp(s - m)
                        w = w / jnp.sum(w, axis=-1, keepdims=True)
                        ctx_ref[b, pl.ds(c * CH, CH), h * DH:(h + 1) * DH] = (
                            jnp.dot(w, v_b[:kmax, h, :],
                                    preferred_element_type=jnp.float32))
                    rdmas[ci].start()
                    project(ci)
            for ci in range(NCH):
                rdmas[ci].wait_send()

        @pl.when(jnp.logical_and(my > 0, my < N_DEV - 1))
        def _forwarders():
            for ci in range(NCH):
                rdmas[ci].wait_recv()
                rdmas[ci].start()
                project(ci)
            for ci in range(NCH):
                rdmas[ci].wait_send()

        @pl.when(my == N_DEV - 1)
        def _last():
            for ci in range(NCH):
                rdmas[ci].wait_recv()
                project(ci)

    return pl.pallas_call(
        body,
        out_shape=jax.ShapeDtypeStruct((B, SQ, D_MODEL), jnp.float32),
        in_specs=[pl.BlockSpec(memory_space=pltpu.VMEM)] * 5,
        out_specs=pl.BlockSpec(memory_space=pltpu.VMEM),
        scratch_shapes=[
            pltpu.VMEM((B, SQ, HQ * DH), jnp.float32),
            pltpu.SemaphoreType.DMA((NCH,)),
            pltpu.SemaphoreType.DMA((NCH,)),
        ],
        compiler_params=pltpu.CompilerParams(collective_id=0),
    )(x, Wq, K_ext, V_ext, Wo)
